# baseline (speedup 1.0000x reference)
import functools

import jax
import jax.numpy as jnp
from jax.experimental import pallas as pl

_H = 16
_NEG = -1e30


def _qkv_kernel(x_ref, w_ref, qkv_ref, sc_ref, *, H):
    xb = x_ref[...].astype(jnp.bfloat16)
    acc = jax.lax.dot_general(
        xb, w_ref[...], (((1,), (1,)), ((), ())),
        preferred_element_type=jnp.float32,
    )
    TM, C3 = acc.shape
    Ch = C3 // (3 * H)
    sq = acc[:, : C3 // 3] ** 2
    s = jnp.sqrt(sq.reshape(TM, H, Ch).sum(axis=2))
    sc_ref[...] = s.T[:, None, :]
    qkv_ref[...] = acc.astype(jnp.bfloat16)


def _thresh_kernel(sc_ref, bias_ref, *, keep):
    s = sc_ref[...].reshape(sc_ref.shape[0], sc_ref.shape[2])
    si = jax.lax.bitcast_convert_type(s, jnp.int32)

    def body(_, lohi):
        lo, hi = lohi
        mid = lo + (hi - lo + 1) // 2
        cnt = jnp.sum((si >= mid).astype(jnp.int32), axis=1, keepdims=True)
        ok = cnt >= keep
        return jnp.where(ok, mid, lo), jnp.where(ok, hi, mid - 1)

    lo = jnp.zeros((si.shape[0], 1), jnp.int32)
    hi = jnp.full((si.shape[0], 1), 0x7F7FFFFF, jnp.int32)
    lo, _ = jax.lax.fori_loop(0, 31, body, (lo, hi))
    bias = jnp.where(si >= lo, 0.0, _NEG).astype(jnp.float32)
    bias_ref[...] = bias[:, None, :]


def _attn_kernel(q_ref, k_ref, v_ref, bias_ref, o_ref, *, scale, Ch, HG):
    outs = []
    for hh in range(HG):
        sl = slice(hh * Ch, (hh + 1) * Ch)
        q = q_ref[:, sl]
        k = k_ref[:, sl]
        logits = jax.lax.dot_general(
            q, k, (((1,), (1,)), ((), ())), preferred_element_type=jnp.float32
        ) * scale
        logits = jnp.clip(logits, -50.0, 50.0) + bias_ref[hh]
        p = jnp.exp(logits)
        w = (p * (1.0 / jnp.sum(p, axis=1, keepdims=True))).astype(jnp.bfloat16)
        outs.append(
            jax.lax.dot_general(
                w, v_ref[:, sl], (((1,), (0,)), ((), ())),
                preferred_element_type=jnp.float32,
            ).astype(jnp.bfloat16)
        )
    o_ref[...] = jnp.concatenate(outs, axis=1)


def _proj_kernel(a_ref, w_ref, b_ref, o_ref):
    acc = jax.lax.dot_general(
        a_ref[...], w_ref[...], (((1,), (1,)), ((), ())),
        preferred_element_type=jnp.float32,
    )
    o_ref[...] = jnp.clip(acc + b_ref[...], -10.0, 10.0)


def kernel(x, W_qkv, W_proj, b_proj):
    B, N, C = x.shape
    H = _H
    Ch = C // H
    BN = B * N
    keep = N // 2
    scale = Ch ** (-0.5)

    x2 = x.reshape(BN, C)
    wq_bf = W_qkv.astype(jnp.bfloat16)
    wp_bf = W_proj.astype(jnp.bfloat16)
    b2 = b_proj.reshape(1, C)

    TM = min(1024, N)
    nrow = BN // TM
    ntile_b = N // TM

    qkv_flat, scores = pl.pallas_call(
        functools.partial(_qkv_kernel, H=H),
        grid=(nrow,),
        in_specs=[
            pl.BlockSpec((TM, C), lambda g: (g, 0)),
            pl.BlockSpec((3 * C, C), lambda g: (0, 0)),
        ],
        out_specs=[
            pl.BlockSpec((TM, 3 * C), lambda g: (g, 0)),
            pl.BlockSpec((H, 1, TM), lambda g: (g // ntile_b, 0, g % ntile_b)),
        ],
        out_shape=[
            jax.ShapeDtypeStruct((BN, 3 * C), jnp.bfloat16),
            jax.ShapeDtypeStruct((B * H, 1, N), jnp.float32),
        ],
    )(x2, wq_bf)

    bias = pl.pallas_call(
        functools.partial(_thresh_kernel, keep=keep),
        in_specs=[pl.BlockSpec((B * H, 1, N), lambda: (0, 0, 0))],
        out_specs=pl.BlockSpec((B * H, 1, N), lambda: (0, 0, 0)),
        out_shape=jax.ShapeDtypeStruct((B * H, 1, N), jnp.float32),
    )(scores)

    TMq = min(1024, N)
    nq = N // TMq
    HG = 16 if H % 16 == 0 else 2
    ng = H // HG
    ncolb = C // (HG * Ch)

    attn_out = pl.pallas_call(
        functools.partial(_attn_kernel, scale=scale, Ch=Ch, HG=HG),
        grid=(B, ng, nq),
        in_specs=[
            pl.BlockSpec((TMq, HG * Ch), lambda b, g, qt: (b * nq + qt, g)),
            pl.BlockSpec((N, HG * Ch), lambda b, g, qt: (b, ncolb + g)),
            pl.BlockSpec((N, HG * Ch), lambda b, g, qt: (b, 2 * ncolb + g)),
            pl.BlockSpec((HG, 1, N), lambda b, g, qt: (b * ng + g, 0, 0)),
        ],
        out_specs=pl.BlockSpec(
            (TMq, HG * Ch), lambda b, g, qt: (b * nq + qt, g)
        ),
        out_shape=jax.ShapeDtypeStruct((BN, C), jnp.bfloat16),
    )(qkv_flat, qkv_flat, qkv_flat, bias)

    out = pl.pallas_call(
        _proj_kernel,
        grid=(nrow,),
        in_specs=[
            pl.BlockSpec((TM, C), lambda g: (g, 0)),
            pl.BlockSpec((C, C), lambda g: (0, 0)),
            pl.BlockSpec((1, C), lambda g: (0, 0)),
        ],
        out_specs=pl.BlockSpec((TM, C), lambda g: (g, 0)),
        out_shape=jax.ShapeDtypeStruct((BN, C), jnp.float32),
    )(attn_out, wp_bf, b2)

    return out.reshape(B, N, C)
